# baseline (speedup 1.0000x reference)
import functools

import jax
import jax.numpy as jnp
from jax.experimental import pallas as pl
from jax.experimental.pallas import tpu as pltpu


def _body(adj_ref, x_ref, w_ref, b_ref, out_ref, deg_ref, s_ref,
          *, blocks: int, block_rows: int):
    i = pl.program_id(0)
    phase1 = i < blocks
    row = (i % blocks) * block_rows

    @pl.when(phase1)
    def _pass1():
        a = adj_ref[...]
        deg_ref[pl.ds(row, block_rows), :] = (
            jnp.sum(a, axis=1, keepdims=True) + 1.0)
        s_ref[pl.ds(row, block_rows), :] = jnp.dot(
            x_ref[...], w_ref[...], preferred_element_type=jnp.float32)

    @pl.when(jnp.logical_not(phase1))
    def _pass2():
        d = jax.lax.rsqrt(deg_ref[...])
        t = d * s_ref[...]
        mv = jnp.dot(adj_ref[...], t, preferred_element_type=jnp.float32)
        d_blk = jax.lax.rsqrt(deg_ref[pl.ds(row, block_rows), :])
        t_blk = d_blk * s_ref[pl.ds(row, block_rows), :]
        out_ref[...] = d_blk * (mv + t_blk) + b_ref[0, 0]


@functools.partial(jax.jit, static_argnames=("block_rows",))
def _run(x, adj, W, b2d, block_rows=512):
    n, f_in = x.shape
    blocks = n // block_rows

    out = pl.pallas_call(
        functools.partial(_body, blocks=blocks, block_rows=block_rows),
        grid=(2 * blocks,),
        in_specs=[
            pl.BlockSpec((block_rows, n), lambda i: (i % blocks, 0)),
            pl.BlockSpec((block_rows, f_in),
                         lambda i: (jnp.where(i < blocks, i, 0), 0)),
            pl.BlockSpec((f_in, 1), lambda i: (0, 0)),
            pl.BlockSpec((1, 1), lambda i: (0, 0)),
        ],
        out_specs=pl.BlockSpec((block_rows, 1), lambda i: (i % blocks, 0)),
        out_shape=jax.ShapeDtypeStruct((n, 1), jnp.float32),
        scratch_shapes=[
            pltpu.VMEM((n, 1), jnp.float32),
            pltpu.VMEM((n, 1), jnp.float32),
        ],
    )(adj, x, W, b2d)
    return out.reshape(1, -1)


def kernel(x, adj, W, b):
    return _run(x, adj, W, b.reshape(1, 1))
